# baseline (speedup 1.0000x reference)
import functools

import jax
import jax.numpy as jnp
from jax import lax
from jax.experimental import pallas as pl
from jax.experimental.pallas import tpu as pltpu
from jax.experimental.pallas import tpu_sc as plsc

B, N, M = 8, 4096, 1024
C1, C2 = 128, 256
H1, H2 = 256, 256
CP = C2 // 2

NBLK = 512
CBLK = 512
P = B * N

NC, NS, L = 2, 16, 16
NW = NC * NS
S = 32

NSPLIT = 2
NB = B // NSPLIT
PS = NB * N
PPW = PS // NW
NCHUNK = PPW // S

MSK_HI = jnp.int32(-65536)


def _splat(vec, pos):
    dnums = lax.GatherDimensionNumbers(
        offset_dims=(), collapsed_slice_dims=(0,), start_index_map=(0,))
    return lax.gather(vec, pos[:, None], dnums, slice_sizes=(1,),
                      mode=lax.GatherScatterMode.PROMISE_IN_BOUNDS)


def _top3_body(ut_ref, kn_ref, idx_ref, w_ref, *, boff):
    b = pl.program_id(0)
    u = ut_ref[0]
    kn = kn_ref[0]
    cross = jnp.dot(kn, u, preferred_element_type=jnp.float32)
    un2 = jnp.sum(u * u, axis=0, keepdims=True)
    kn2 = jnp.sum(kn * kn, axis=1, keepdims=True)
    d2 = jnp.maximum(kn2 + un2 - 2.0 * cross, 0.0)

    col = lax.broadcasted_iota(jnp.int32, (M, NBLK), 0)
    keys = (lax.bitcast_convert_type(d2, jnp.int32) & jnp.int32(~1023)) | col

    mins = []
    for k in range(3):
        mn = jnp.min(keys, axis=0, keepdims=True)
        mins.append(mn)
        if k < 2:
            keys = jnp.where(keys == mn, jnp.int32(0x7FFFFFFF), keys)

    idxs = [mn & jnp.int32(1023) for mn in mins]
    d2s = [lax.bitcast_convert_type(mn & jnp.int32(~1023), jnp.float32)
           for mn in mins]
    recips = [1.0 / (d + 1e-8) for d in d2s]
    norm = recips[0] + recips[1] + recips[2]
    ws = [r / norm for r in recips]

    zi = jnp.zeros((1, NBLK), jnp.int32)
    zf = jnp.zeros((1, NBLK), jnp.float32)
    goff = (b + boff) * M
    idx_ref[0, 0] = jnp.concatenate(
        [idxs[0] + goff, idxs[1] + goff, idxs[2] + goff, zi, zi, zi, zi, zi],
        axis=0)
    w_ref[0, 0] = jnp.concatenate([ws[0], ws[1], ws[2], zf, zf, zf, zf, zf],
                                  axis=0)


def _top3(ut8, kn8, boff):
    wpb = N // NBLK
    return pl.pallas_call(
        functools.partial(_top3_body, boff=boff),
        grid=(NB, wpb),
        in_specs=[
            pl.BlockSpec((1, 8, NBLK), lambda b, i: (b + boff, 0, i)),
            pl.BlockSpec((1, M, 8), lambda b, i: (b + boff, 0, 0)),
        ],
        out_specs=[
            pl.BlockSpec((1, 1, 8, NBLK), lambda b, i: (b, i, 0, 0)),
            pl.BlockSpec((1, 1, 8, NBLK), lambda b, i: (b, i, 0, 0)),
        ],
        out_shape=[
            jax.ShapeDtypeStruct((NB, wpb, 8, NBLK), jnp.int32),
            jax.ShapeDtypeStruct((NB, wpb, 8, NBLK), jnp.float32),
        ],
    )(ut8, kn8)


def _interp_sc(idx_w, w_w, table):
    mesh = plsc.VectorSubcoreMesh(core_axis_name="c", subcore_axis_name="s")

    @functools.partial(
        pl.kernel,
        mesh=mesh,
        out_type=jax.ShapeDtypeStruct((PS, C2), jnp.float32),
        scratch_types=[
            pltpu.VMEM((8, NCHUNK, S), jnp.int32),
            pltpu.VMEM((3, PPW), jnp.float32),
            pltpu.VMEM((2, 3, S, C2), jnp.float32),
            pltpu.VMEM((2, S, C2), jnp.float32),
            pltpu.SemaphoreType.DMA,
            pltpu.SemaphoreType.DMA,
            pltpu.SemaphoreType.DMA,
            pltpu.SemaphoreType.DMA,
        ],
    )
    def body(idx_hbm, w_hbm, table_hbm, out_hbm, idx_v, w_v, rows_v, out_v,
             semg0, semg1, semo0, semo1):
        wid = lax.axis_index("s") * NC + lax.axis_index("c")
        base = wid * PPW
        semg = [semg0, semg1]
        semo = [semo0, semo1]
        pltpu.sync_copy(idx_hbm.at[wid], idx_v)
        pltpu.sync_copy(w_hbm.at[wid, pl.ds(0, 3)], w_v)

        def start_gather(c, buf):
            for k in range(3):
                pltpu.async_copy(table_hbm.at[idx_v.at[k, c]],
                                 rows_v.at[buf, k], semg[buf])

        def wait_gather(c, buf):
            for k in range(3):
                pltpu.make_async_copy(table_hbm.at[idx_v.at[k, c]],
                                      rows_v.at[buf, k], semg[buf]).wait()

        def start_out(c, buf):
            pltpu.async_copy(out_v.at[buf],
                             out_hbm.at[pl.ds(base + c * S, S)], semo[buf])

        def wait_out(buf):
            pltpu.make_async_copy(out_v.at[buf],
                                  out_hbm.at[pl.ds(base, S)],
                                  semo[buf]).wait()

        def compute(c, buf):
            start = c * S

            def point_body(p, carry2):
                al = start + pl.multiple_of((p // L) * L, L)
                pos = jnp.full((L,), p % L, jnp.int32)
                w0 = _splat(w_v[0, pl.ds(al, L)], pos)
                w1 = _splat(w_v[1, pl.ds(al, L)], pos)
                w2 = _splat(w_v[2, pl.ds(al, L)], pos)
                for j in range(C2 // L):
                    sl = pl.ds(j * L, L)
                    acc = (w0 * rows_v[buf, 0, p, sl]
                           + w1 * rows_v[buf, 1, p, sl]
                           + w2 * rows_v[buf, 2, p, sl])
                    out_v[buf, p, sl] = acc
                return carry2

            lax.fori_loop(0, S, point_body, 0)

        start_gather(0, 0)
        nhalf = NCHUNK // 2

        def pair_body(g, carry):
            c0 = g * 2
            wait_gather(c0, 0)
            start_gather(c0 + 1, 1)

            @pl.when(g > 0)
            def _():
                wait_out(0)

            compute(c0, 0)
            start_out(c0, 0)

            wait_gather(c0 + 1, 1)

            @pl.when(g < nhalf - 1)
            def _():
                start_gather(c0 + 2, 0)

            @pl.when(g > 0)
            def _():
                wait_out(1)

            compute(c0 + 1, 1)
            start_out(c0 + 1, 1)
            return carry

        lax.fori_loop(0, nhalf, pair_body, 0)
        wait_out(0)
        wait_out(1)

    return body(idx_w, w_w, table)


def _mlp_half_body(it_ref, uf_ref, w1a_ref, w1b_ref, b1_ref, w2_ref,
                   b2_ref, out_ref):
    x = (jnp.dot(it_ref[...], w1a_ref[...], preferred_element_type=jnp.float32)
         + jnp.dot(uf_ref[...], w1b_ref[...],
                   preferred_element_type=jnp.float32)
         + b1_ref[...])
    x = jnp.maximum(x, 0.0)
    y = (jnp.dot(x, w2_ref[...], preferred_element_type=jnp.float32)
         + b2_ref[...])
    out_ref[...] = jnp.maximum(y, 0.0)


def _mlp_half(interp, uf, w1a, w1b, b1f, w2f, b2f, boff, prev=None):
    nblk = PS // CBLK

    def _pb(prev_ref, it_ref, uf_ref, w1a_ref, w1b_ref, b1_ref, w2_ref,
            b2_ref, out_ref):
        _mlp_half_body(it_ref, uf_ref, w1a_ref, w1b_ref, b1_ref, w2_ref,
                       b2_ref, out_ref)

    in_specs = [
        pl.BlockSpec((CBLK, C2), lambda i: (i, 0)),
        pl.BlockSpec((CBLK, C1), lambda i: (i + boff, 0)),
        pl.BlockSpec((C2, H1), lambda i: (0, 0)),
        pl.BlockSpec((C1, H1), lambda i: (0, 0)),
        pl.BlockSpec((1, H1), lambda i: (0, 0)),
        pl.BlockSpec((H1, H2), lambda i: (0, 0)),
        pl.BlockSpec((1, H2), lambda i: (0, 0)),
    ]
    args = (interp, uf, w1a, w1b, b1f, w2f, b2f)
    kwargs = {}
    body = _mlp_half_body
    if prev is not None:
        in_specs = [pl.BlockSpec(memory_space=pl.ANY)] + in_specs
        args = (prev,) + args
        kwargs = dict(input_output_aliases={0: 0})
        body = _pb
    return pl.pallas_call(
        body,
        grid=(nblk,),
        in_specs=in_specs,
        out_specs=pl.BlockSpec((CBLK, H2), lambda i: (i + boff, 0)),
        out_shape=jax.ShapeDtypeStruct((P, H2), jnp.float32),
        **kwargs,
    )(*args)


def kernel(unknown, known, unknow_feats, known_feats, grouped_xyz, inds,
           W1, b1, gamma1, beta1, W2, b2, gamma2, beta2):
    ut8 = jnp.zeros((B, 8, N), jnp.float32).at[:, :3, :].set(
        jnp.transpose(unknown, (0, 2, 1)))
    kn8 = jnp.zeros((B, M, 8), jnp.float32).at[:, :, :3].set(known)

    table = known_feats.reshape(B * M, C2)

    s1 = gamma1 / jnp.sqrt(1.0 + 1e-3)
    s2 = gamma2 / jnp.sqrt(1.0 + 1e-3)
    w1f = W1 * s1[None, :]
    b1f = (b1 * s1 + beta1).reshape(1, H1)
    w2f = W2 * s2[None, :]
    b2f = (b2 * s2 + beta2).reshape(1, H2)
    w1a = w1f[:C2]
    w1b = w1f[C2:]

    interps = []
    for s in range(NSPLIT):
        idx_pad, w_pad = _top3(ut8, kn8, s * NB)
        idx_w = idx_pad.reshape(NW, 8, NCHUNK, S)
        w_w = w_pad.reshape(NW, 8, PPW)
        interps.append(_interp_sc(idx_w, w_w, table))

    uf = unknow_feats.reshape(P, C1)
    nblk = PS // CBLK
    out = _mlp_half(interps[0], uf, w1a, w1b, b1f, w2f, b2f, 0)
    out = _mlp_half(interps[1], uf, w1a, w1b, b1f, w2f, b2f, nblk,
                    prev=out)
    return out.reshape(B, N, H2)


# speedup vs baseline: 1.0530x; 1.0530x over previous
import functools

import jax
import jax.numpy as jnp
from jax import lax
from jax.experimental import pallas as pl
from jax.experimental.pallas import tpu as pltpu
from jax.experimental.pallas import tpu_sc as plsc

B, N, M = 8, 4096, 1024
C1, C2 = 128, 256
H1, H2 = 256, 256
CP = C2 // 2

NBLK = 512
CBLK = 1024
P = B * N

NC, NS, L = 2, 16, 16
NW = NC * NS
S = 32

NSPLIT = 2
NB = B // NSPLIT
PS = NB * N
PPW = PS // NW
NCHUNK = PPW // S

MSK_HI = jnp.int32(-65536)


def _splat(vec, pos):
    dnums = lax.GatherDimensionNumbers(
        offset_dims=(), collapsed_slice_dims=(0,), start_index_map=(0,))
    return lax.gather(vec, pos[:, None], dnums, slice_sizes=(1,),
                      mode=lax.GatherScatterMode.PROMISE_IN_BOUNDS)


def _top3_body(ut_ref, kn_ref, idx_ref, w_ref, *, boff):
    b = pl.program_id(0)
    u = ut_ref[0]
    kn = kn_ref[0]
    cross = jnp.dot(kn, u, preferred_element_type=jnp.float32)
    un2 = jnp.sum(u * u, axis=0, keepdims=True)
    kn2 = jnp.sum(kn * kn, axis=1, keepdims=True)
    d2 = jnp.maximum(kn2 + un2 - 2.0 * cross, 0.0)

    col = lax.broadcasted_iota(jnp.int32, (M, NBLK), 0)
    keys = (lax.bitcast_convert_type(d2, jnp.int32) & jnp.int32(~1023)) | col

    mins = []
    for k in range(3):
        mn = jnp.min(keys, axis=0, keepdims=True)
        mins.append(mn)
        if k < 2:
            keys = jnp.where(keys == mn, jnp.int32(0x7FFFFFFF), keys)

    idxs = [mn & jnp.int32(1023) for mn in mins]
    d2s = [lax.bitcast_convert_type(mn & jnp.int32(~1023), jnp.float32)
           for mn in mins]
    recips = [1.0 / (d + 1e-8) for d in d2s]
    norm = recips[0] + recips[1] + recips[2]
    ws = [r / norm for r in recips]

    zi = jnp.zeros((1, NBLK), jnp.int32)
    zf = jnp.zeros((1, NBLK), jnp.float32)
    goff = (b + boff) * M
    idx_ref[0, 0] = jnp.concatenate(
        [idxs[0] + goff, idxs[1] + goff, idxs[2] + goff, zi, zi, zi, zi, zi],
        axis=0)
    w_ref[0, 0] = jnp.concatenate([ws[0], ws[1], ws[2], zf, zf, zf, zf, zf],
                                  axis=0)


def _top3(ut8, kn8, boff):
    wpb = N // NBLK
    return pl.pallas_call(
        functools.partial(_top3_body, boff=boff),
        grid=(NB, wpb),
        in_specs=[
            pl.BlockSpec((1, 8, NBLK), lambda b, i: (b + boff, 0, i)),
            pl.BlockSpec((1, M, 8), lambda b, i: (b + boff, 0, 0)),
        ],
        out_specs=[
            pl.BlockSpec((1, 1, 8, NBLK), lambda b, i: (b, i, 0, 0)),
            pl.BlockSpec((1, 1, 8, NBLK), lambda b, i: (b, i, 0, 0)),
        ],
        out_shape=[
            jax.ShapeDtypeStruct((NB, wpb, 8, NBLK), jnp.int32),
            jax.ShapeDtypeStruct((NB, wpb, 8, NBLK), jnp.float32),
        ],
    )(ut8, kn8)


def _interp_sc(idx_w, w_w, table):
    mesh = plsc.VectorSubcoreMesh(core_axis_name="c", subcore_axis_name="s")

    @functools.partial(
        pl.kernel,
        mesh=mesh,
        out_type=jax.ShapeDtypeStruct((PS, C2), jnp.float32),
        scratch_types=[
            pltpu.VMEM((8, NCHUNK, S), jnp.int32),
            pltpu.VMEM((3, PPW), jnp.float32),
            pltpu.VMEM((2, 3, S, C2), jnp.float32),
            pltpu.VMEM((2, S, C2), jnp.float32),
            pltpu.SemaphoreType.DMA,
            pltpu.SemaphoreType.DMA,
            pltpu.SemaphoreType.DMA,
            pltpu.SemaphoreType.DMA,
        ],
    )
    def body(idx_hbm, w_hbm, table_hbm, out_hbm, idx_v, w_v, rows_v, out_v,
             semg0, semg1, semo0, semo1):
        wid = lax.axis_index("s") * NC + lax.axis_index("c")
        base = wid * PPW
        semg = [semg0, semg1]
        semo = [semo0, semo1]
        pltpu.sync_copy(idx_hbm.at[wid], idx_v)
        pltpu.sync_copy(w_hbm.at[wid, pl.ds(0, 3)], w_v)

        def start_gather(c, buf):
            for k in range(3):
                pltpu.async_copy(table_hbm.at[idx_v.at[k, c]],
                                 rows_v.at[buf, k], semg[buf])

        def wait_gather(c, buf):
            for k in range(3):
                pltpu.make_async_copy(table_hbm.at[idx_v.at[k, c]],
                                      rows_v.at[buf, k], semg[buf]).wait()

        def start_out(c, buf):
            pltpu.async_copy(out_v.at[buf],
                             out_hbm.at[pl.ds(base + c * S, S)], semo[buf])

        def wait_out(buf):
            pltpu.make_async_copy(out_v.at[buf],
                                  out_hbm.at[pl.ds(base, S)],
                                  semo[buf]).wait()

        def compute(c, buf):
            start = c * S

            def point_body(p, carry2):
                al = start + pl.multiple_of((p // L) * L, L)
                pos = jnp.full((L,), p % L, jnp.int32)
                w0 = _splat(w_v[0, pl.ds(al, L)], pos)
                w1 = _splat(w_v[1, pl.ds(al, L)], pos)
                w2 = _splat(w_v[2, pl.ds(al, L)], pos)
                for j in range(C2 // L):
                    sl = pl.ds(j * L, L)
                    acc = (w0 * rows_v[buf, 0, p, sl]
                           + w1 * rows_v[buf, 1, p, sl]
                           + w2 * rows_v[buf, 2, p, sl])
                    out_v[buf, p, sl] = acc
                return carry2

            lax.fori_loop(0, S, point_body, 0)

        start_gather(0, 0)
        nhalf = NCHUNK // 2

        def pair_body(g, carry):
            c0 = g * 2
            wait_gather(c0, 0)
            start_gather(c0 + 1, 1)

            @pl.when(g > 0)
            def _():
                wait_out(0)

            compute(c0, 0)
            start_out(c0, 0)

            wait_gather(c0 + 1, 1)

            @pl.when(g < nhalf - 1)
            def _():
                start_gather(c0 + 2, 0)

            @pl.when(g > 0)
            def _():
                wait_out(1)

            compute(c0 + 1, 1)
            start_out(c0 + 1, 1)
            return carry

        lax.fori_loop(0, nhalf, pair_body, 0)
        wait_out(0)
        wait_out(1)

    return body(idx_w, w_w, table)


def _mlp_half_body(it_ref, uf_ref, w1a_ref, w1b_ref, b1_ref, w2_ref,
                   b2_ref, out_ref):
    x = (jnp.dot(it_ref[...], w1a_ref[...], preferred_element_type=jnp.float32)
         + jnp.dot(uf_ref[...], w1b_ref[...],
                   preferred_element_type=jnp.float32)
         + b1_ref[...])
    x = jnp.maximum(x, 0.0)
    y = (jnp.dot(x, w2_ref[...], preferred_element_type=jnp.float32)
         + b2_ref[...])
    out_ref[...] = jnp.maximum(y, 0.0)


def _mlp_half(interp, uf, w1a, w1b, b1f, w2f, b2f, boff, prev=None):
    nblk = PS // CBLK

    def _pb(prev_ref, it_ref, uf_ref, w1a_ref, w1b_ref, b1_ref, w2_ref,
            b2_ref, out_ref):
        _mlp_half_body(it_ref, uf_ref, w1a_ref, w1b_ref, b1_ref, w2_ref,
                       b2_ref, out_ref)

    in_specs = [
        pl.BlockSpec((CBLK, C2), lambda i: (i, 0)),
        pl.BlockSpec((CBLK, C1), lambda i: (i + boff, 0)),
        pl.BlockSpec((C2, H1), lambda i: (0, 0)),
        pl.BlockSpec((C1, H1), lambda i: (0, 0)),
        pl.BlockSpec((1, H1), lambda i: (0, 0)),
        pl.BlockSpec((H1, H2), lambda i: (0, 0)),
        pl.BlockSpec((1, H2), lambda i: (0, 0)),
    ]
    args = (interp, uf, w1a, w1b, b1f, w2f, b2f)
    kwargs = {}
    body = _mlp_half_body
    if prev is not None:
        in_specs = [pl.BlockSpec(memory_space=pl.ANY)] + in_specs
        args = (prev,) + args
        kwargs = dict(input_output_aliases={0: 0})
        body = _pb
    return pl.pallas_call(
        body,
        grid=(nblk,),
        in_specs=in_specs,
        out_specs=pl.BlockSpec((CBLK, H2), lambda i: (i + boff, 0)),
        out_shape=jax.ShapeDtypeStruct((P, H2), jnp.float32),
        **kwargs,
    )(*args)


def kernel(unknown, known, unknow_feats, known_feats, grouped_xyz, inds,
           W1, b1, gamma1, beta1, W2, b2, gamma2, beta2):
    ut8 = jnp.zeros((B, 8, N), jnp.float32).at[:, :3, :].set(
        jnp.transpose(unknown, (0, 2, 1)))
    kn8 = jnp.zeros((B, M, 8), jnp.float32).at[:, :, :3].set(known)

    table = known_feats.reshape(B * M, C2)

    s1 = gamma1 / jnp.sqrt(1.0 + 1e-3)
    s2 = gamma2 / jnp.sqrt(1.0 + 1e-3)
    w1f = W1 * s1[None, :]
    b1f = (b1 * s1 + beta1).reshape(1, H1)
    w2f = W2 * s2[None, :]
    b2f = (b2 * s2 + beta2).reshape(1, H2)
    w1a = w1f[:C2]
    w1b = w1f[C2:]

    interps = []
    for s in range(NSPLIT):
        idx_pad, w_pad = _top3(ut8, kn8, s * NB)
        idx_w = idx_pad.reshape(NW, 8, NCHUNK, S)
        w_w = w_pad.reshape(NW, 8, PPW)
        interps.append(_interp_sc(idx_w, w_w, table))

    uf = unknow_feats.reshape(P, C1)
    nblk = PS // CBLK
    out = _mlp_half(interps[0], uf, w1a, w1b, b1f, w2f, b2f, 0)
    out = _mlp_half(interps[1], uf, w1a, w1b, b1f, w2f, b2f, nblk,
                    prev=out)
    return out.reshape(B, N, H2)


# speedup vs baseline: 1.0770x; 1.0228x over previous
import functools

import jax
import jax.numpy as jnp
from jax import lax
from jax.experimental import pallas as pl
from jax.experimental.pallas import tpu as pltpu
from jax.experimental.pallas import tpu_sc as plsc

B, N, M = 8, 4096, 1024
C1, C2 = 128, 256
H1, H2 = 256, 256
CP = C2 // 2

NBLK = 512
CBLK = 2048
P = B * N

NC, NS, L = 2, 16, 16
NW = NC * NS
S = 32

NSPLIT = 2
NB = B // NSPLIT
PS = NB * N
PPW = PS // NW
NCHUNK = PPW // S

MSK_HI = jnp.int32(-65536)


def _splat(vec, pos):
    dnums = lax.GatherDimensionNumbers(
        offset_dims=(), collapsed_slice_dims=(0,), start_index_map=(0,))
    return lax.gather(vec, pos[:, None], dnums, slice_sizes=(1,),
                      mode=lax.GatherScatterMode.PROMISE_IN_BOUNDS)


def _top3_body(ut_ref, kn_ref, idx_ref, w_ref, *, boff):
    b = pl.program_id(0)
    u = ut_ref[0]
    kn = kn_ref[0]
    cross = jnp.dot(kn, u, preferred_element_type=jnp.float32)
    un2 = jnp.sum(u * u, axis=0, keepdims=True)
    kn2 = jnp.sum(kn * kn, axis=1, keepdims=True)
    d2 = jnp.maximum(kn2 + un2 - 2.0 * cross, 0.0)

    col = lax.broadcasted_iota(jnp.int32, (M, NBLK), 0)
    keys = (lax.bitcast_convert_type(d2, jnp.int32) & jnp.int32(~1023)) | col

    mins = []
    for k in range(3):
        mn = jnp.min(keys, axis=0, keepdims=True)
        mins.append(mn)
        if k < 2:
            keys = jnp.where(keys == mn, jnp.int32(0x7FFFFFFF), keys)

    idxs = [mn & jnp.int32(1023) for mn in mins]
    d2s = [lax.bitcast_convert_type(mn & jnp.int32(~1023), jnp.float32)
           for mn in mins]
    recips = [1.0 / (d + 1e-8) for d in d2s]
    norm = recips[0] + recips[1] + recips[2]
    ws = [r / norm for r in recips]

    zi = jnp.zeros((1, NBLK), jnp.int32)
    zf = jnp.zeros((1, NBLK), jnp.float32)
    goff = (b + boff) * M
    idx_ref[0, 0] = jnp.concatenate(
        [idxs[0] + goff, idxs[1] + goff, idxs[2] + goff, zi, zi, zi, zi, zi],
        axis=0)
    w_ref[0, 0] = jnp.concatenate([ws[0], ws[1], ws[2], zf, zf, zf, zf, zf],
                                  axis=0)


def _top3(ut8, kn8, boff):
    wpb = N // NBLK
    return pl.pallas_call(
        functools.partial(_top3_body, boff=boff),
        grid=(NB, wpb),
        in_specs=[
            pl.BlockSpec((1, 8, NBLK), lambda b, i: (b + boff, 0, i)),
            pl.BlockSpec((1, M, 8), lambda b, i: (b + boff, 0, 0)),
        ],
        out_specs=[
            pl.BlockSpec((1, 1, 8, NBLK), lambda b, i: (b, i, 0, 0)),
            pl.BlockSpec((1, 1, 8, NBLK), lambda b, i: (b, i, 0, 0)),
        ],
        out_shape=[
            jax.ShapeDtypeStruct((NB, wpb, 8, NBLK), jnp.int32),
            jax.ShapeDtypeStruct((NB, wpb, 8, NBLK), jnp.float32),
        ],
    )(ut8, kn8)


def _interp_sc(idx_w, w_w, table):
    mesh = plsc.VectorSubcoreMesh(core_axis_name="c", subcore_axis_name="s")

    @functools.partial(
        pl.kernel,
        mesh=mesh,
        out_type=jax.ShapeDtypeStruct((PS, C2), jnp.float32),
        scratch_types=[
            pltpu.VMEM((8, NCHUNK, S), jnp.int32),
            pltpu.VMEM((3, PPW), jnp.float32),
            pltpu.VMEM((2, 3, S, C2), jnp.float32),
            pltpu.VMEM((2, S, C2), jnp.float32),
            pltpu.SemaphoreType.DMA,
            pltpu.SemaphoreType.DMA,
            pltpu.SemaphoreType.DMA,
            pltpu.SemaphoreType.DMA,
        ],
    )
    def body(idx_hbm, w_hbm, table_hbm, out_hbm, idx_v, w_v, rows_v, out_v,
             semg0, semg1, semo0, semo1):
        wid = lax.axis_index("s") * NC + lax.axis_index("c")
        base = wid * PPW
        semg = [semg0, semg1]
        semo = [semo0, semo1]
        pltpu.sync_copy(idx_hbm.at[wid], idx_v)
        pltpu.sync_copy(w_hbm.at[wid, pl.ds(0, 3)], w_v)

        def start_gather(c, buf):
            for k in range(3):
                pltpu.async_copy(table_hbm.at[idx_v.at[k, c]],
                                 rows_v.at[buf, k], semg[buf])

        def wait_gather(c, buf):
            for k in range(3):
                pltpu.make_async_copy(table_hbm.at[idx_v.at[k, c]],
                                      rows_v.at[buf, k], semg[buf]).wait()

        def start_out(c, buf):
            pltpu.async_copy(out_v.at[buf],
                             out_hbm.at[pl.ds(base + c * S, S)], semo[buf])

        def wait_out(buf):
            pltpu.make_async_copy(out_v.at[buf],
                                  out_hbm.at[pl.ds(base, S)],
                                  semo[buf]).wait()

        def compute(c, buf):
            start = c * S

            def point_body(p, carry2):
                al = start + pl.multiple_of((p // L) * L, L)
                pos = jnp.full((L,), p % L, jnp.int32)
                w0 = _splat(w_v[0, pl.ds(al, L)], pos)
                w1 = _splat(w_v[1, pl.ds(al, L)], pos)
                w2 = _splat(w_v[2, pl.ds(al, L)], pos)
                for j in range(C2 // L):
                    sl = pl.ds(j * L, L)
                    acc = (w0 * rows_v[buf, 0, p, sl]
                           + w1 * rows_v[buf, 1, p, sl]
                           + w2 * rows_v[buf, 2, p, sl])
                    out_v[buf, p, sl] = acc
                return carry2

            lax.fori_loop(0, S, point_body, 0)

        start_gather(0, 0)
        nhalf = NCHUNK // 2

        def pair_body(g, carry):
            c0 = g * 2
            wait_gather(c0, 0)
            start_gather(c0 + 1, 1)

            @pl.when(g > 0)
            def _():
                wait_out(0)

            compute(c0, 0)
            start_out(c0, 0)

            wait_gather(c0 + 1, 1)

            @pl.when(g < nhalf - 1)
            def _():
                start_gather(c0 + 2, 0)

            @pl.when(g > 0)
            def _():
                wait_out(1)

            compute(c0 + 1, 1)
            start_out(c0 + 1, 1)
            return carry

        lax.fori_loop(0, nhalf, pair_body, 0)
        wait_out(0)
        wait_out(1)

    return body(idx_w, w_w, table)


def _mlp_half_body(it_ref, uf_ref, w1a_ref, w1b_ref, b1_ref, w2_ref,
                   b2_ref, out_ref):
    x = (jnp.dot(it_ref[...], w1a_ref[...], preferred_element_type=jnp.float32)
         + jnp.dot(uf_ref[...], w1b_ref[...],
                   preferred_element_type=jnp.float32)
         + b1_ref[...])
    x = jnp.maximum(x, 0.0)
    y = (jnp.dot(x, w2_ref[...], preferred_element_type=jnp.float32)
         + b2_ref[...])
    out_ref[...] = jnp.maximum(y, 0.0)


def _mlp_half(interp, uf, w1a, w1b, b1f, w2f, b2f, boff, prev=None):
    nblk = PS // CBLK

    def _pb(prev_ref, it_ref, uf_ref, w1a_ref, w1b_ref, b1_ref, w2_ref,
            b2_ref, out_ref):
        _mlp_half_body(it_ref, uf_ref, w1a_ref, w1b_ref, b1_ref, w2_ref,
                       b2_ref, out_ref)

    in_specs = [
        pl.BlockSpec((CBLK, C2), lambda i: (i, 0)),
        pl.BlockSpec((CBLK, C1), lambda i: (i + boff, 0)),
        pl.BlockSpec((C2, H1), lambda i: (0, 0)),
        pl.BlockSpec((C1, H1), lambda i: (0, 0)),
        pl.BlockSpec((1, H1), lambda i: (0, 0)),
        pl.BlockSpec((H1, H2), lambda i: (0, 0)),
        pl.BlockSpec((1, H2), lambda i: (0, 0)),
    ]
    args = (interp, uf, w1a, w1b, b1f, w2f, b2f)
    kwargs = {}
    body = _mlp_half_body
    if prev is not None:
        in_specs = [pl.BlockSpec(memory_space=pl.ANY)] + in_specs
        args = (prev,) + args
        kwargs = dict(input_output_aliases={0: 0})
        body = _pb
    return pl.pallas_call(
        body,
        grid=(nblk,),
        in_specs=in_specs,
        out_specs=pl.BlockSpec((CBLK, H2), lambda i: (i + boff, 0)),
        out_shape=jax.ShapeDtypeStruct((P, H2), jnp.float32),
        **kwargs,
    )(*args)


def kernel(unknown, known, unknow_feats, known_feats, grouped_xyz, inds,
           W1, b1, gamma1, beta1, W2, b2, gamma2, beta2):
    ut8 = jnp.zeros((B, 8, N), jnp.float32).at[:, :3, :].set(
        jnp.transpose(unknown, (0, 2, 1)))
    kn8 = jnp.zeros((B, M, 8), jnp.float32).at[:, :, :3].set(known)

    table = known_feats.reshape(B * M, C2)

    s1 = gamma1 / jnp.sqrt(1.0 + 1e-3)
    s2 = gamma2 / jnp.sqrt(1.0 + 1e-3)
    w1f = W1 * s1[None, :]
    b1f = (b1 * s1 + beta1).reshape(1, H1)
    w2f = W2 * s2[None, :]
    b2f = (b2 * s2 + beta2).reshape(1, H2)
    w1a = w1f[:C2]
    w1b = w1f[C2:]

    interps = []
    for s in range(NSPLIT):
        idx_pad, w_pad = _top3(ut8, kn8, s * NB)
        idx_w = idx_pad.reshape(NW, 8, NCHUNK, S)
        w_w = w_pad.reshape(NW, 8, PPW)
        interps.append(_interp_sc(idx_w, w_w, table))

    uf = unknow_feats.reshape(P, C1)
    nblk = PS // CBLK
    out = _mlp_half(interps[0], uf, w1a, w1b, b1f, w2f, b2f, 0)
    out = _mlp_half(interps[1], uf, w1a, w1b, b1f, w2f, b2f, nblk,
                    prev=out)
    return out.reshape(B, N, H2)


# speedup vs baseline: 1.0876x; 1.0099x over previous
import functools

import jax
import jax.numpy as jnp
from jax import lax
from jax.experimental import pallas as pl
from jax.experimental.pallas import tpu as pltpu
from jax.experimental.pallas import tpu_sc as plsc

B, N, M = 8, 4096, 1024
C1, C2 = 128, 256
H1, H2 = 256, 256
CP = C2 // 2

NBLK = 512
CBLK = 4096
P = B * N

NC, NS, L = 2, 16, 16
NW = NC * NS
S = 32

NSPLIT = 2
NB = B // NSPLIT
PS = NB * N
PPW = PS // NW
NCHUNK = PPW // S

MSK_HI = jnp.int32(-65536)


def _splat(vec, pos):
    dnums = lax.GatherDimensionNumbers(
        offset_dims=(), collapsed_slice_dims=(0,), start_index_map=(0,))
    return lax.gather(vec, pos[:, None], dnums, slice_sizes=(1,),
                      mode=lax.GatherScatterMode.PROMISE_IN_BOUNDS)


def _top3_body(ut_ref, kn_ref, idx_ref, w_ref, *, boff):
    b = pl.program_id(0)
    u = ut_ref[0]
    kn = kn_ref[0]
    cross = jnp.dot(kn, u, preferred_element_type=jnp.float32)
    un2 = jnp.sum(u * u, axis=0, keepdims=True)
    kn2 = jnp.sum(kn * kn, axis=1, keepdims=True)
    d2 = jnp.maximum(kn2 + un2 - 2.0 * cross, 0.0)

    col = lax.broadcasted_iota(jnp.int32, (M, NBLK), 0)
    keys = (lax.bitcast_convert_type(d2, jnp.int32) & jnp.int32(~1023)) | col

    mins = []
    for k in range(3):
        mn = jnp.min(keys, axis=0, keepdims=True)
        mins.append(mn)
        if k < 2:
            keys = jnp.where(keys == mn, jnp.int32(0x7FFFFFFF), keys)

    idxs = [mn & jnp.int32(1023) for mn in mins]
    d2s = [lax.bitcast_convert_type(mn & jnp.int32(~1023), jnp.float32)
           for mn in mins]
    recips = [1.0 / (d + 1e-8) for d in d2s]
    norm = recips[0] + recips[1] + recips[2]
    ws = [r / norm for r in recips]

    zi = jnp.zeros((1, NBLK), jnp.int32)
    zf = jnp.zeros((1, NBLK), jnp.float32)
    goff = (b + boff) * M
    idx_ref[0, 0] = jnp.concatenate(
        [idxs[0] + goff, idxs[1] + goff, idxs[2] + goff, zi, zi, zi, zi, zi],
        axis=0)
    w_ref[0, 0] = jnp.concatenate([ws[0], ws[1], ws[2], zf, zf, zf, zf, zf],
                                  axis=0)


def _top3(ut8, kn8, boff):
    wpb = N // NBLK
    return pl.pallas_call(
        functools.partial(_top3_body, boff=boff),
        grid=(NB, wpb),
        in_specs=[
            pl.BlockSpec((1, 8, NBLK), lambda b, i: (b + boff, 0, i)),
            pl.BlockSpec((1, M, 8), lambda b, i: (b + boff, 0, 0)),
        ],
        out_specs=[
            pl.BlockSpec((1, 1, 8, NBLK), lambda b, i: (b, i, 0, 0)),
            pl.BlockSpec((1, 1, 8, NBLK), lambda b, i: (b, i, 0, 0)),
        ],
        out_shape=[
            jax.ShapeDtypeStruct((NB, wpb, 8, NBLK), jnp.int32),
            jax.ShapeDtypeStruct((NB, wpb, 8, NBLK), jnp.float32),
        ],
    )(ut8, kn8)


def _interp_sc(idx_w, w_w, table):
    mesh = plsc.VectorSubcoreMesh(core_axis_name="c", subcore_axis_name="s")

    @functools.partial(
        pl.kernel,
        mesh=mesh,
        out_type=jax.ShapeDtypeStruct((PS, C2), jnp.float32),
        scratch_types=[
            pltpu.VMEM((8, NCHUNK, S), jnp.int32),
            pltpu.VMEM((3, PPW), jnp.float32),
            pltpu.VMEM((2, 3, S, C2), jnp.float32),
            pltpu.VMEM((2, S, C2), jnp.float32),
            pltpu.SemaphoreType.DMA,
            pltpu.SemaphoreType.DMA,
            pltpu.SemaphoreType.DMA,
            pltpu.SemaphoreType.DMA,
        ],
    )
    def body(idx_hbm, w_hbm, table_hbm, out_hbm, idx_v, w_v, rows_v, out_v,
             semg0, semg1, semo0, semo1):
        wid = lax.axis_index("s") * NC + lax.axis_index("c")
        base = wid * PPW
        semg = [semg0, semg1]
        semo = [semo0, semo1]
        pltpu.sync_copy(idx_hbm.at[wid], idx_v)
        pltpu.sync_copy(w_hbm.at[wid, pl.ds(0, 3)], w_v)

        def start_gather(c, buf):
            for k in range(3):
                pltpu.async_copy(table_hbm.at[idx_v.at[k, c]],
                                 rows_v.at[buf, k], semg[buf])

        def wait_gather(c, buf):
            for k in range(3):
                pltpu.make_async_copy(table_hbm.at[idx_v.at[k, c]],
                                      rows_v.at[buf, k], semg[buf]).wait()

        def start_out(c, buf):
            pltpu.async_copy(out_v.at[buf],
                             out_hbm.at[pl.ds(base + c * S, S)], semo[buf])

        def wait_out(buf):
            pltpu.make_async_copy(out_v.at[buf],
                                  out_hbm.at[pl.ds(base, S)],
                                  semo[buf]).wait()

        def compute(c, buf):
            start = c * S

            def point_body(p, carry2):
                al = start + pl.multiple_of((p // L) * L, L)
                pos = jnp.full((L,), p % L, jnp.int32)
                w0 = _splat(w_v[0, pl.ds(al, L)], pos)
                w1 = _splat(w_v[1, pl.ds(al, L)], pos)
                w2 = _splat(w_v[2, pl.ds(al, L)], pos)
                for j in range(C2 // L):
                    sl = pl.ds(j * L, L)
                    acc = (w0 * rows_v[buf, 0, p, sl]
                           + w1 * rows_v[buf, 1, p, sl]
                           + w2 * rows_v[buf, 2, p, sl])
                    out_v[buf, p, sl] = acc
                return carry2

            lax.fori_loop(0, S, point_body, 0)

        start_gather(0, 0)
        nhalf = NCHUNK // 2

        def pair_body(g, carry):
            c0 = g * 2
            wait_gather(c0, 0)
            start_gather(c0 + 1, 1)

            @pl.when(g > 0)
            def _():
                wait_out(0)

            compute(c0, 0)
            start_out(c0, 0)

            wait_gather(c0 + 1, 1)

            @pl.when(g < nhalf - 1)
            def _():
                start_gather(c0 + 2, 0)

            @pl.when(g > 0)
            def _():
                wait_out(1)

            compute(c0 + 1, 1)
            start_out(c0 + 1, 1)
            return carry

        lax.fori_loop(0, nhalf, pair_body, 0)
        wait_out(0)
        wait_out(1)

    return body(idx_w, w_w, table)


def _mlp_half_body(it_ref, uf_ref, w1a_ref, w1b_ref, b1_ref, w2_ref,
                   b2_ref, out_ref):
    x = (jnp.dot(it_ref[...], w1a_ref[...], preferred_element_type=jnp.float32)
         + jnp.dot(uf_ref[...], w1b_ref[...],
                   preferred_element_type=jnp.float32)
         + b1_ref[...])
    x = jnp.maximum(x, 0.0)
    y = (jnp.dot(x, w2_ref[...], preferred_element_type=jnp.float32)
         + b2_ref[...])
    out_ref[...] = jnp.maximum(y, 0.0)


def _mlp_half(interp, uf, w1a, w1b, b1f, w2f, b2f, boff, prev=None):
    nblk = PS // CBLK

    def _pb(prev_ref, it_ref, uf_ref, w1a_ref, w1b_ref, b1_ref, w2_ref,
            b2_ref, out_ref):
        _mlp_half_body(it_ref, uf_ref, w1a_ref, w1b_ref, b1_ref, w2_ref,
                       b2_ref, out_ref)

    in_specs = [
        pl.BlockSpec((CBLK, C2), lambda i: (i, 0)),
        pl.BlockSpec((CBLK, C1), lambda i: (i + boff, 0)),
        pl.BlockSpec((C2, H1), lambda i: (0, 0)),
        pl.BlockSpec((C1, H1), lambda i: (0, 0)),
        pl.BlockSpec((1, H1), lambda i: (0, 0)),
        pl.BlockSpec((H1, H2), lambda i: (0, 0)),
        pl.BlockSpec((1, H2), lambda i: (0, 0)),
    ]
    args = (interp, uf, w1a, w1b, b1f, w2f, b2f)
    kwargs = {}
    body = _mlp_half_body
    if prev is not None:
        in_specs = [pl.BlockSpec(memory_space=pl.ANY)] + in_specs
        args = (prev,) + args
        kwargs = dict(input_output_aliases={0: 0})
        body = _pb
    return pl.pallas_call(
        body,
        grid=(nblk,),
        in_specs=in_specs,
        out_specs=pl.BlockSpec((CBLK, H2), lambda i: (i + boff, 0)),
        out_shape=jax.ShapeDtypeStruct((P, H2), jnp.float32),
        **kwargs,
    )(*args)


def kernel(unknown, known, unknow_feats, known_feats, grouped_xyz, inds,
           W1, b1, gamma1, beta1, W2, b2, gamma2, beta2):
    ut8 = jnp.zeros((B, 8, N), jnp.float32).at[:, :3, :].set(
        jnp.transpose(unknown, (0, 2, 1)))
    kn8 = jnp.zeros((B, M, 8), jnp.float32).at[:, :, :3].set(known)

    table = known_feats.reshape(B * M, C2)

    s1 = gamma1 / jnp.sqrt(1.0 + 1e-3)
    s2 = gamma2 / jnp.sqrt(1.0 + 1e-3)
    w1f = W1 * s1[None, :]
    b1f = (b1 * s1 + beta1).reshape(1, H1)
    w2f = W2 * s2[None, :]
    b2f = (b2 * s2 + beta2).reshape(1, H2)
    w1a = w1f[:C2]
    w1b = w1f[C2:]

    interps = []
    for s in range(NSPLIT):
        idx_pad, w_pad = _top3(ut8, kn8, s * NB)
        idx_w = idx_pad.reshape(NW, 8, NCHUNK, S)
        w_w = w_pad.reshape(NW, 8, PPW)
        interps.append(_interp_sc(idx_w, w_w, table))

    uf = unknow_feats.reshape(P, C1)
    nblk = PS // CBLK
    out = _mlp_half(interps[0], uf, w1a, w1b, b1f, w2f, b2f, 0)
    out = _mlp_half(interps[1], uf, w1a, w1b, b1f, w2f, b2f, nblk,
                    prev=out)
    return out.reshape(B, N, H2)
